# baseline (device time: 44296 ns/iter reference)
import jax
import jax.numpy as jnp
from jax import lax
from jax.experimental import pallas as pl
from jax.experimental.pallas import tpu as pltpu

N_DEV = 4
B, SQ, D = 4, 256, 1024
HQL = 8
DH = 128
KVL = 2
NKV = KVL * DH
SCALE = 0.08838834764831843
BF = jnp.bfloat16
F32 = jnp.float32


def kernel(x, Wq, Wo, Wk, Wv):
    my = lax.axis_index("i")
    Wk_loc = lax.dynamic_slice(Wk, (0, my * NKV), (D, NKV))
    Wv_loc = lax.dynamic_slice(Wv, (0, my * NKV), (D, NKV))

    def body(x_hbm, wq_hbm, wo_hbm, wk_hbm, wv_hbm, out_ref,
             x_vm, wq_vm, wk_vm, wv_vm, wo_vm,
             wqkv_bf, wo_bf,
             attn_ref, rs_send, rs_recv, ag_send,
             copy_sems, rs_send_sems, rs_recv_sems,
             ag_send_sems, ag_recv_sems):
        my_pos = lax.axis_index("i")
        peers = [(my_pos + 1 + j) % N_DEV for j in range(N_DEV - 1)]
        chunks = peers + [my_pos]

        cp_x = [
            pltpu.make_async_copy(
                x_hbm.at[pl.ds(chunks[j], 1)], x_vm.at[j], copy_sems.at[j])
            for j in range(N_DEV)
        ]
        cp_wk = pltpu.make_async_copy(wk_hbm, wk_vm, copy_sems.at[4])
        cp_wv = pltpu.make_async_copy(wv_hbm, wv_vm, copy_sems.at[5])
        cp_wq = pltpu.make_async_copy(wq_hbm, wq_vm, copy_sems.at[6])
        cp_wo = pltpu.make_async_copy(wo_hbm, wo_vm, copy_sems.at[7])
        cp_x[0].start()
        cp_wk.start()
        cp_wv.start()
        cp_wq.start()
        cp_wo.start()
        for j in range(1, N_DEV):
            cp_x[j].start()

        def compute_partial(j):
            cp_x[j].wait()
            xb = x_vm[j].reshape(SQ, D).astype(BF)
            if j == 0:
                cp_wq.wait()
                wqkv_bf[:, :D] = wq_vm[...].astype(BF)
                cp_wk.wait()
                cp_wv.wait()
                wqkv_bf[:, D:D + NKV] = wk_vm[...].astype(BF)
                wqkv_bf[:, D + NKV:] = wv_vm[...].astype(BF)
            qkv = jnp.dot(xb, wqkv_bf[...],
                          preferred_element_type=F32).astype(BF)
            for h in range(HQL):
                g = h // 4
                qh = qkv[:, h * DH:(h + 1) * DH]
                kh = qkv[:, D + g * DH:D + (g + 1) * DH]
                vh = qkv[:, D + NKV + g * DH:D + NKV + (g + 1) * DH]
                s = lax.dot_general(
                    qh, kh, (((1,), (1,)), ((), ())),
                    preferred_element_type=F32,
                ) * SCALE
                p = jnp.exp(s)
                l = jnp.sum(p, axis=-1, keepdims=True)
                oh = jnp.dot(p.astype(BF), vh, preferred_element_type=F32)
                attn_ref[:, h * DH:(h + 1) * DH] = (oh / l).astype(BF)
            if j == 0:
                cp_wo.wait()
                wo_bf[...] = wo_vm[...].astype(BF)
            return jnp.dot(attn_ref[...], wo_bf[...],
                           preferred_element_type=F32)

        barrier_sem = pltpu.get_barrier_semaphore()
        for p in peers:
            pl.semaphore_signal(
                barrier_sem, inc=1,
                device_id=(p,), device_id_type=pl.DeviceIdType.MESH,
            )

        rdmas = []
        for j in range(N_DEV - 1):
            rs_send[j, :, :] = compute_partial(j).astype(BF)
            if j == 0:
                pl.semaphore_wait(barrier_sem, N_DEV - 1)
            r = pltpu.make_async_remote_copy(
                src_ref=rs_send.at[j],
                dst_ref=rs_recv.at[N_DEV - 2 - j],
                send_sem=rs_send_sems.at[j],
                recv_sem=rs_recv_sems.at[N_DEV - 2 - j],
                device_id=(peers[j],),
                device_id_type=pl.DeviceIdType.MESH,
            )
            r.start()
            rdmas.append(r)

        local = compute_partial(N_DEV - 1)

        acc = local
        for s in range(N_DEV - 1):
            w = pltpu.make_async_remote_copy(
                src_ref=rs_send.at[0], dst_ref=rs_recv.at[s],
                send_sem=rs_send_sems.at[0], recv_sem=rs_recv_sems.at[s],
                device_id=(my_pos,), device_id_type=pl.DeviceIdType.MESH,
            )
            w.wait_recv()
            acc = acc + rs_recv[s, :, :].astype(F32)
        ag_send[0, :, :] = acc.astype(BF)
        out_ref[pl.ds(my_pos, 1), :, :] = ag_send[...]

        for j in range(N_DEV - 1):
            a = pltpu.make_async_remote_copy(
                src_ref=ag_send,
                dst_ref=out_ref.at[pl.ds(my_pos, 1)],
                send_sem=ag_send_sems.at[j],
                recv_sem=ag_recv_sems.at[N_DEV - 2 - j],
                device_id=(peers[j],),
                device_id_type=pl.DeviceIdType.MESH,
            )
            a.start()
            rdmas.append(a)
        for s in range(N_DEV - 1):
            c = (my_pos + 1 + s) % N_DEV
            w = pltpu.make_async_remote_copy(
                src_ref=ag_send, dst_ref=out_ref.at[pl.ds(c, 1)],
                send_sem=ag_send_sems.at[0], recv_sem=ag_recv_sems.at[s],
                device_id=(my_pos,), device_id_type=pl.DeviceIdType.MESH,
            )
            w.wait_recv()

        for r in rdmas:
            r.wait_send()

    return pl.pallas_call(
        body,
        out_shape=jax.ShapeDtypeStruct((B, SQ, D), BF),
        in_specs=[pl.BlockSpec(memory_space=pltpu.MemorySpace.HBM)] * 5,
        out_specs=pl.BlockSpec(memory_space=pltpu.VMEM),
        scratch_shapes=[
            pltpu.VMEM((N_DEV, 1, SQ, D), F32),
            pltpu.VMEM((D, D), F32),
            pltpu.VMEM((D, NKV), F32),
            pltpu.VMEM((D, NKV), F32),
            pltpu.VMEM((D, D), F32),
            pltpu.VMEM((D, D + 2 * NKV), BF),
            pltpu.VMEM((D, D), BF),
            pltpu.VMEM((SQ, D), BF),
            pltpu.VMEM((N_DEV - 1, SQ, D), BF),
            pltpu.VMEM((N_DEV - 1, SQ, D), BF),
            pltpu.VMEM((1, SQ, D), BF),
            pltpu.SemaphoreType.DMA((8,)),
            pltpu.SemaphoreType.DMA((N_DEV - 1,)),
            pltpu.SemaphoreType.DMA((N_DEV - 1,)),
            pltpu.SemaphoreType.DMA((N_DEV - 1,)),
            pltpu.SemaphoreType.DMA((N_DEV - 1,)),
        ],
        compiler_params=pltpu.CompilerParams(collective_id=0),
    )(x, Wq, Wo, Wk_loc, Wv_loc)


# device time: 43539 ns/iter; 1.0174x vs baseline; 1.0174x over previous
import jax
import jax.numpy as jnp
from jax import lax
from jax.experimental import pallas as pl
from jax.experimental.pallas import tpu as pltpu

N_DEV = 4
B, SQ, D = 4, 256, 1024
HQL = 8
DH = 128
KVL = 2
NKV = KVL * DH
SCALE = 0.08838834764831843
BF = jnp.bfloat16
F32 = jnp.float32


def kernel(x, Wq, Wo, Wk, Wv):
    my = lax.axis_index("i")
    Wk_loc = lax.dynamic_slice(Wk, (0, my * NKV), (D, NKV))
    Wv_loc = lax.dynamic_slice(Wv, (0, my * NKV), (D, NKV))

    def body(x_ref, wq_ref, wo_ref, wk_ref, wv_ref, out_ref,
             wqkv_bf, attn_ref, rs_send, rs_recv, ag_send,
             rs_send_sems, rs_recv_sems, ag_send_sems, ag_recv_sems):
        my_pos = lax.axis_index("i")
        peers = [(my_pos + 1 + j) % N_DEV for j in range(N_DEV - 1)]

        wqkv_bf[:, :D] = wq_ref[...].astype(BF)
        wqkv_bf[:, D:D + NKV] = wk_ref[...].astype(BF)
        wqkv_bf[:, D + NKV:] = wv_ref[...].astype(BF)
        wo = wo_ref[...].astype(BF)

        def compute_partial(b):
            xb = x_ref[pl.ds(b, 1), :, :].reshape(SQ, D).astype(BF)
            qkv = jnp.dot(xb, wqkv_bf[...],
                          preferred_element_type=F32).astype(BF)
            for h in range(HQL):
                g = h // 4
                qh = qkv[:, h * DH:(h + 1) * DH]
                kh = qkv[:, D + g * DH:D + (g + 1) * DH]
                vh = qkv[:, D + NKV + g * DH:D + NKV + (g + 1) * DH]
                s = lax.dot_general(
                    qh, kh, (((1,), (1,)), ((), ())),
                    preferred_element_type=F32,
                ) * SCALE
                p = jnp.exp(s)
                l = jnp.sum(p, axis=-1, keepdims=True)
                oh = jnp.dot(p.astype(BF), vh, preferred_element_type=F32)
                attn_ref[:, h * DH:(h + 1) * DH] = (oh / l).astype(BF)
            return jnp.dot(attn_ref[...], wo, preferred_element_type=F32)

        barrier_sem = pltpu.get_barrier_semaphore()
        for p in peers:
            pl.semaphore_signal(
                barrier_sem, inc=1,
                device_id=(p,), device_id_type=pl.DeviceIdType.MESH,
            )

        rdmas = []
        for j in range(N_DEV - 1):
            rs_send[j, :, :] = compute_partial(peers[j]).astype(BF)
            if j == 0:
                pl.semaphore_wait(barrier_sem, N_DEV - 1)
            r = pltpu.make_async_remote_copy(
                src_ref=rs_send.at[j],
                dst_ref=rs_recv.at[N_DEV - 2 - j],
                send_sem=rs_send_sems.at[j],
                recv_sem=rs_recv_sems.at[N_DEV - 2 - j],
                device_id=(peers[j],),
                device_id_type=pl.DeviceIdType.MESH,
            )
            r.start()
            rdmas.append(r)

        local = compute_partial(my_pos)

        acc = local
        for s in range(N_DEV - 1):
            w = pltpu.make_async_remote_copy(
                src_ref=rs_send.at[0], dst_ref=rs_recv.at[s],
                send_sem=rs_send_sems.at[0], recv_sem=rs_recv_sems.at[s],
                device_id=(my_pos,), device_id_type=pl.DeviceIdType.MESH,
            )
            w.wait_recv()
            acc = acc + rs_recv[s, :, :].astype(F32)
        ag_send[0, :, :] = acc.astype(BF)
        out_ref[pl.ds(my_pos, 1), :, :] = ag_send[...]

        for j in range(N_DEV - 1):
            a = pltpu.make_async_remote_copy(
                src_ref=ag_send,
                dst_ref=out_ref.at[pl.ds(my_pos, 1)],
                send_sem=ag_send_sems.at[j],
                recv_sem=ag_recv_sems.at[N_DEV - 2 - j],
                device_id=(peers[j],),
                device_id_type=pl.DeviceIdType.MESH,
            )
            a.start()
            rdmas.append(a)
        for s in range(N_DEV - 1):
            c = (my_pos + 1 + s) % N_DEV
            w = pltpu.make_async_remote_copy(
                src_ref=ag_send, dst_ref=out_ref.at[pl.ds(c, 1)],
                send_sem=ag_send_sems.at[0], recv_sem=ag_recv_sems.at[s],
                device_id=(my_pos,), device_id_type=pl.DeviceIdType.MESH,
            )
            w.wait_recv()

        for r in rdmas:
            r.wait_send()

    return pl.pallas_call(
        body,
        out_shape=jax.ShapeDtypeStruct((B, SQ, D), BF),
        in_specs=[pl.BlockSpec(memory_space=pltpu.VMEM)] * 5,
        out_specs=pl.BlockSpec(memory_space=pltpu.VMEM),
        scratch_shapes=[
            pltpu.VMEM((D, D + 2 * NKV), BF),
            pltpu.VMEM((SQ, D), BF),
            pltpu.VMEM((N_DEV - 1, SQ, D), BF),
            pltpu.VMEM((N_DEV - 1, SQ, D), BF),
            pltpu.VMEM((1, SQ, D), BF),
            pltpu.SemaphoreType.DMA((N_DEV - 1,)),
            pltpu.SemaphoreType.DMA((N_DEV - 1,)),
            pltpu.SemaphoreType.DMA((N_DEV - 1,)),
            pltpu.SemaphoreType.DMA((N_DEV - 1,)),
        ],
        compiler_params=pltpu.CompilerParams(collective_id=0),
    )(x, Wq, Wo, Wk_loc, Wv_loc)


# device time: 18268 ns/iter; 2.4248x vs baseline; 2.3833x over previous
import jax
import jax.numpy as jnp
from jax import lax
from jax.experimental import pallas as pl
from jax.experimental.pallas import tpu as pltpu

N_DEV = 4
B, SQ, D = 4, 256, 1024
HQ2 = SQ // 2
HQL = 8
DH = 128
KVL = 2
NKV = KVL * DH
SCALE = 0.08838834764831843
BF = jnp.bfloat16
F32 = jnp.float32


def kernel(x, Wq, Wo, Wk, Wv):
    my = lax.axis_index("i")
    Wk_loc = lax.dynamic_slice(Wk, (0, my * NKV), (D, NKV))
    Wv_loc = lax.dynamic_slice(Wv, (0, my * NKV), (D, NKV))

    def body(x_ref, wq_ref, wo_ref, wk_ref, wv_ref, out_ref,
             wqkv_bf, kv_cache, attn_ref, rs_send, rs_recv, ag_send,
             rs_send_sems, rs_recv_sems, ag_send_sems, ag_recv_sems):
        my_pos = lax.axis_index("i")
        peers = [(my_pos + 1 + j) % N_DEV for j in range(N_DEV - 1)]
        batches = peers + [my_pos]

        wqkv_bf[:, :D] = wq_ref[...].astype(BF)
        wqkv_bf[:, D:D + NKV] = wk_ref[...].astype(BF)
        wqkv_bf[:, D + NKV:] = wv_ref[...].astype(BF)
        wo = wo_ref[...].astype(BF)

        def compute_partial(j, e):
            b = batches[j]
            if e == 0:
                xf = x_ref[pl.ds(b, 1), :, :].reshape(SQ, D).astype(BF)
                kv_cache[j, :, :] = jnp.dot(
                    xf, wqkv_bf[:, D:], preferred_element_type=F32
                ).astype(BF)
            xh = x_ref[pl.ds(b, 1), pl.ds(e * HQ2, HQ2), :].reshape(
                HQ2, D).astype(BF)
            q = jnp.dot(xh, wqkv_bf[:, :D],
                        preferred_element_type=F32).astype(BF)
            for h in range(HQL):
                g = h // 4
                qh = q[:, h * DH:(h + 1) * DH]
                kh = kv_cache[j, :, g * DH:(g + 1) * DH]
                vh = kv_cache[j, :, NKV + g * DH:NKV + (g + 1) * DH]
                s = lax.dot_general(
                    qh, kh, (((1,), (1,)), ((), ())),
                    preferred_element_type=F32,
                ) * SCALE
                p = jnp.exp(s)
                l = jnp.sum(p, axis=-1, keepdims=True)
                oh = jnp.dot(p.astype(BF), vh, preferred_element_type=F32)
                attn_ref[:, h * DH:(h + 1) * DH] = (oh / l).astype(BF)
            return jnp.dot(attn_ref[...], wo, preferred_element_type=F32)

        barrier_sem = pltpu.get_barrier_semaphore()
        for p in peers:
            pl.semaphore_signal(
                barrier_sem, inc=1,
                device_id=(p,), device_id_type=pl.DeviceIdType.MESH,
            )

        rdmas = []

        def half_phase(e):
            for j in range(N_DEV - 1):
                rs_send[e, j, :, :] = compute_partial(j, e).astype(BF)
                if e == 0 and j == 0:
                    pl.semaphore_wait(barrier_sem, N_DEV - 1)
                r = pltpu.make_async_remote_copy(
                    src_ref=rs_send.at[e, j],
                    dst_ref=rs_recv.at[e, N_DEV - 2 - j],
                    send_sem=rs_send_sems.at[e, j],
                    recv_sem=rs_recv_sems.at[e, N_DEV - 2 - j],
                    device_id=(peers[j],),
                    device_id_type=pl.DeviceIdType.MESH,
                )
                r.start()
                rdmas.append(r)

            acc = compute_partial(N_DEV - 1, e)
            for s in range(N_DEV - 1):
                w = pltpu.make_async_remote_copy(
                    src_ref=rs_send.at[e, 0], dst_ref=rs_recv.at[e, s],
                    send_sem=rs_send_sems.at[e, 0],
                    recv_sem=rs_recv_sems.at[e, s],
                    device_id=(my_pos,),
                    device_id_type=pl.DeviceIdType.MESH,
                )
                w.wait_recv()
                acc = acc + rs_recv[e, s, :, :].astype(F32)
            ag_send[e, :, :] = acc.astype(BF)
            out_ref[pl.ds(my_pos, 1), pl.ds(e * HQ2, HQ2), :] = (
                ag_send[pl.ds(e, 1), :, :])

            for j in range(N_DEV - 1):
                a = pltpu.make_async_remote_copy(
                    src_ref=ag_send.at[pl.ds(e, 1)],
                    dst_ref=out_ref.at[pl.ds(my_pos, 1),
                                       pl.ds(e * HQ2, HQ2)],
                    send_sem=ag_send_sems.at[e, j],
                    recv_sem=ag_recv_sems.at[e, N_DEV - 2 - j],
                    device_id=(peers[j],),
                    device_id_type=pl.DeviceIdType.MESH,
                )
                a.start()
                rdmas.append(a)

        half_phase(0)
        half_phase(1)

        for e in range(2):
            for s in range(N_DEV - 1):
                c = (my_pos + 1 + s) % N_DEV
                w = pltpu.make_async_remote_copy(
                    src_ref=ag_send.at[pl.ds(e, 1)],
                    dst_ref=out_ref.at[pl.ds(c, 1), pl.ds(e * HQ2, HQ2)],
                    send_sem=ag_send_sems.at[e, 0],
                    recv_sem=ag_recv_sems.at[e, s],
                    device_id=(my_pos,),
                    device_id_type=pl.DeviceIdType.MESH,
                )
                w.wait_recv()

        for r in rdmas:
            r.wait_send()

    return pl.pallas_call(
        body,
        out_shape=jax.ShapeDtypeStruct((B, SQ, D), BF),
        in_specs=[pl.BlockSpec(memory_space=pltpu.VMEM)] * 5,
        out_specs=pl.BlockSpec(memory_space=pltpu.VMEM),
        scratch_shapes=[
            pltpu.VMEM((D, D + 2 * NKV), BF),
            pltpu.VMEM((N_DEV, SQ, 2 * NKV), BF),
            pltpu.VMEM((HQ2, D), BF),
            pltpu.VMEM((2, N_DEV - 1, HQ2, D), BF),
            pltpu.VMEM((2, N_DEV - 1, HQ2, D), BF),
            pltpu.VMEM((2, HQ2, D), BF),
            pltpu.SemaphoreType.DMA((2, N_DEV - 1)),
            pltpu.SemaphoreType.DMA((2, N_DEV - 1)),
            pltpu.SemaphoreType.DMA((2, N_DEV - 1)),
            pltpu.SemaphoreType.DMA((2, N_DEV - 1)),
        ],
        compiler_params=pltpu.CompilerParams(collective_id=0),
    )(x, Wq, Wo, Wk_loc, Wv_loc)
